# baseline (speedup 1.0000x reference)
import jax
import jax.numpy as jnp
from jax import lax
from jax.experimental import pallas as pl
from jax.experimental.pallas import tpu as pltpu
from jax.experimental.pallas import tpu_sc as plsc

BATCH = 4096
FIELDS = 26
DIM = 64
VPACK = 2
WPACK = 8
NC = 2
NS = 16
NW = NC * NS
B_PER_W = BATCH // NW
IDX_PER_W = B_PER_W * FIELDS
GROUP = 16
E_PER_C = 4
CHUNK = E_PER_C * FIELDS
N_CHUNKS = B_PER_W // E_PER_C
PROW = VPACK * DIM
NJ = DIM // GROUP


def _fm_body(x_hbm, v2_hbm, w8_hbm, b_hbm, out_hbm,
             idx_v, idxh_v, idx8_v, rows0, rows1, wrow0, wrow1, out_v, b_v,
             sem0, sem1):
    wid = lax.axis_index("s") * NC + lax.axis_index("c")

    pltpu.sync_copy(x_hbm.at[pl.ds(wid * IDX_PER_W, IDX_PER_W)], idx_v)
    pltpu.sync_copy(b_hbm, b_v)

    SH_UNROLL = 4

    def shift_step(i, _):
        for u in range(SH_UNROLL):
            o = (i * SH_UNROLL + u) * GROUP
            xv = idx_v[pl.ds(o, GROUP)]
            idxh_v[pl.ds(o, GROUP)] = lax.shift_right_logical(xv, 1)
            idx8_v[pl.ds(o, GROUP)] = lax.shift_right_logical(xv, 3)
        return 0
    lax.fori_loop(0, IDX_PER_W // (GROUP * SH_UNROLL), shift_step, 0)

    b_s = b_v[...]
    lane = lax.iota(jnp.int32, GROUP)
    seven = jnp.full((GROUP,), 7, jnp.int32)
    one = jnp.full((GROUP,), 1, jnp.int32)
    half = jnp.float32(0.5)
    zf = jnp.zeros((GROUP,), jnp.float32)
    zi = jnp.zeros((GROUP,), jnp.int32)

    def issue(c, rows_r, wrow_r, sem):
        start = pl.multiple_of(c * CHUNK, 8)
        cpv = pltpu.async_copy(v2_hbm.at[idxh_v.at[pl.ds(start, CHUNK)]],
                               rows_r, sem)
        cpw = pltpu.async_copy(w8_hbm.at[idx8_v.at[pl.ds(start, CHUNK)]],
                               wrow_r, sem)
        return cpv, cpw

    def drain(rows_r, wrow_r, sem):
        pltpu.make_async_copy(v2_hbm.at[pl.ds(0, CHUNK)], rows_r, sem).wait()
        pltpu.make_async_copy(w8_hbm.at[pl.ds(0, CHUNK)], wrow_r, sem).wait()

    def consume(c, rows_r, wrow_r, carry):
        interv, linv = carry
        lanebase = (lax.rem(c, 4) * E_PER_C).astype(jnp.int32)

        def elem_step(e, cr):
            iv, lv = cr
            flat0 = c * CHUNK + e * FIELDS
            acc_a = [zf] * NJ
            acc_b = [zf] * NJ
            sq_a = [zf] * NJ
            sq_b = [zf] * NJ
            linacc = zf
            for f in range(FIELDS):
                xi = plsc.load_gather(idx_v, [zi + (flat0 + f)])
                podd = jnp.bitwise_and(xi, one) > 0
                col = jnp.bitwise_and(xi, seven)
                wrow = zi + (e * FIELDS + f)
                linacc = linacc + plsc.load_gather(wrow_r, [wrow, col])
                for j in range(NJ):
                    lo = rows_r[e * FIELDS + f, pl.ds(j * GROUP, GROUP)]
                    hi = rows_r[e * FIELDS + f, pl.ds(DIM + j * GROUP, GROUP)]
                    v = jnp.where(podd, hi, lo)
                    if f % 2 == 0:
                        acc_a[j] = acc_a[j] + v
                        sq_a[j] = sq_a[j] + v * v
                    else:
                        acc_b[j] = acc_b[j] + v
                        sq_b[j] = sq_b[j] + v * v
            h = zf
            for j in range(NJ):
                a = acc_a[j] + acc_b[j]
                h = h + (a * a - (sq_a[j] + sq_b[j]))
            sel = lane == (lanebase + e)
            iv = jnp.where(sel, jnp.sum(h), iv)
            lv = jnp.where(sel, linacc, lv)
            return iv, lv

        return lax.fori_loop(0, E_PER_C, elem_step, (interv, linv))

    issue(0, rows0, wrow0, sem0)
    issue(1, rows1, wrow1, sem1)

    def pair_step(i, carry):
        c0 = i * 2
        drain(rows0, wrow0, sem0)
        carry = consume(c0, rows0, wrow0, carry)

        @pl.when(i < N_CHUNKS // 2 - 1)
        def _():
            issue(c0 + 2, rows0, wrow0, sem0)

        drain(rows1, wrow1, sem1)
        carry = consume(c0 + 1, rows1, wrow1, carry)

        @pl.when(i < N_CHUNKS // 2 - 1)
        def _():
            issue(c0 + 3, rows1, wrow1, sem1)

        interv, linv = carry

        @pl.when(lax.rem(i, 2) == 1)
        def _():
            z = linv + b_s + half * interv
            gstart = (i // 2) * GROUP
            out_v[pl.ds(pl.multiple_of(gstart, 8), GROUP)] = (
                1.0 / (1.0 + jnp.exp(-z)))

        done = lax.rem(i, 2) == 1
        interv = jnp.where(done, zf, interv)
        linv = jnp.where(done, zf, linv)
        return interv, linv

    lax.fori_loop(0, N_CHUNKS // 2, pair_step, (zf, zf))

    pltpu.sync_copy(out_v, out_hbm.at[pl.ds(wid * B_PER_W, B_PER_W)])


def kernel(X, y, V, w, b):
    xf = X.astype(jnp.int32).reshape(BATCH * FIELDS)
    v2 = V.reshape(V.shape[0] // VPACK, PROW)
    w8 = w.reshape(w.shape[0] // WPACK, WPACK)
    b16 = jnp.broadcast_to(b.astype(jnp.float32), (GROUP,))
    mesh = plsc.VectorSubcoreMesh(core_axis_name="c", subcore_axis_name="s",
                                  num_cores=NC, num_subcores=NS)
    fm = pl.kernel(
        _fm_body,
        out_type=jax.ShapeDtypeStruct((BATCH,), jnp.float32),
        mesh=mesh,
        scratch_types=[
            pltpu.VMEM((IDX_PER_W,), jnp.int32),
            pltpu.VMEM((IDX_PER_W,), jnp.int32),
            pltpu.VMEM((IDX_PER_W,), jnp.int32),
            pltpu.VMEM((CHUNK, PROW), jnp.float32),
            pltpu.VMEM((CHUNK, PROW), jnp.float32),
            pltpu.VMEM((CHUNK, WPACK), jnp.float32),
            pltpu.VMEM((CHUNK, WPACK), jnp.float32),
            pltpu.VMEM((B_PER_W,), jnp.float32),
            pltpu.VMEM((GROUP,), jnp.float32),
            pltpu.SemaphoreType.DMA,
            pltpu.SemaphoreType.DMA,
        ],
        compiler_params=pltpu.CompilerParams(needs_layout_passes=False,
                                             use_tc_tiling_on_sc=False),
    )
    y_pred = fm(xf, v2, w8, b16).reshape(BATCH, 1)
    y_true = y.reshape(BATCH, 1)
    return (y_true, y_pred)


# speedup vs baseline: 1.5310x; 1.5310x over previous
import jax
import jax.numpy as jnp
from jax import lax
from jax.experimental import pallas as pl
from jax.experimental.pallas import tpu as pltpu
from jax.experimental.pallas import tpu_sc as plsc

BATCH = 4096
FIELDS = 26
DIM = 64
WPACK = 8
NC = 2
NS = 16
NW = NC * NS
B_PER_W = BATCH // NW
IDX_PER_W = B_PER_W * FIELDS
GROUP = 16
CHUNK = GROUP * FIELDS // 4
N_GROUPS = B_PER_W // GROUP
ROWS_PER_G = GROUP * FIELDS
D_UNROLL = 2


def _treesum(vals):
    vals = list(vals)
    while len(vals) > 1:
        nxt = [vals[i] + vals[i + 1] for i in range(0, len(vals) - 1, 2)]
        if len(vals) % 2:
            nxt.append(vals[-1])
        vals = nxt
    return vals[0]


def _fm_body(x_hbm, v_hbm, w8_hbm, b_hbm, out_hbm,
             idx_v, idx8_v, rows0, rows1, rows2, wrow0, wrow1, wrow2,
             out_v, b_v, sem0, sem1, sem2):
    wid = lax.axis_index("s") * NC + lax.axis_index("c")

    pltpu.sync_copy(x_hbm.at[pl.ds(wid * IDX_PER_W, IDX_PER_W)], idx_v)
    pltpu.sync_copy(b_hbm, b_v)

    b_s = b_v[...]
    lane = lax.iota(jnp.int32, GROUP)
    rowbase = lane * FIELDS
    seven = jnp.full((GROUP,), 7, jnp.int32)
    half = jnp.float32(0.5)
    zf = jnp.zeros((GROUP,), jnp.float32)
    zi = jnp.zeros((GROUP,), jnp.int32)

    NBUF = 3
    bufs = [(rows0, wrow0, sem0), (rows1, wrow1, sem1), (rows2, wrow2, sem2)]

    def issue_v(g):
        rows_r, _, sem = bufs[g % NBUF]
        return [pltpu.async_copy(
            v_hbm.at[idx_v.at[pl.ds((g * 4 + j) * CHUNK, CHUNK)]],
            rows_r.at[pl.ds(j * CHUNK, CHUNK)], sem) for j in range(4)]

    def issue_w(g):
        _, wrow_r, sem = bufs[g % NBUF]
        return [pltpu.async_copy(
            w8_hbm.at[idx8_v.at[pl.ds((g * 4 + j) * CHUNK, CHUNK)]],
            wrow_r.at[pl.ds(j * CHUNK, CHUNK)], sem) for j in range(4)]

    pending = {0: issue_v(0), 1: issue_v(1)}

    SH_UNROLL = 4

    def shift_step(i, _):
        for u in range(SH_UNROLL):
            o = (i * SH_UNROLL + u) * GROUP
            xv = idx_v[pl.ds(o, GROUP)]
            idx8_v[pl.ds(o, GROUP)] = lax.shift_right_logical(xv, 3)
        return 0
    lax.fori_loop(0, IDX_PER_W // (GROUP * SH_UNROLL), shift_step, 0)

    pending[0] += issue_w(0)
    pending[1] += issue_w(1)

    for g in range(N_GROUPS):
        if g + 2 < N_GROUPS:
            pending[g + 2] = issue_v(g + 2) + issue_w(g + 2)
        for cp in pending.pop(g):
            cp.wait()

        rows_r, wrow_r, _ = bufs[g % NBUF]

        NJ = DIM // GROUP

        def elem_step(l, z):
            row0 = l * FIELDS
            acc_a = [zf] * NJ
            acc_b = [zf] * NJ
            sq_a = [zf] * NJ
            sq_b = [zf] * NJ
            for f in range(FIELDS):
                r = row0 + f
                for j in range(NJ):
                    v = rows_r[r, pl.ds(j * GROUP, GROUP)]
                    if f % 2 == 0:
                        acc_a[j] = acc_a[j] + v
                        sq_a[j] = sq_a[j] + v * v
                    else:
                        acc_b[j] = acc_b[j] + v
                        sq_b[j] = sq_b[j] + v * v
            h = zf
            for j in range(NJ):
                a = acc_a[j] + acc_b[j]
                h = h + (a * a - (sq_a[j] + sq_b[j]))
            inter_s = jnp.sum(h)
            return jnp.where(lane == l, inter_s, z)

        inter_v = lax.fori_loop(0, GROUP, elem_step, zf)

        wvals = []
        for f in range(FIELDS):
            xi = plsc.load_gather(idx_v, [rowbase + (g * ROWS_PER_G + f)])
            col = jnp.bitwise_and(xi, seven)
            wvals.append(plsc.load_gather(wrow_r, [rowbase + f, col]))
        lin = _treesum(wvals)

        z = lin + b_s + half * inter_v
        out_v[pl.ds(g * GROUP, GROUP)] = 1.0 / (1.0 + jnp.exp(-z))

    pltpu.sync_copy(out_v, out_hbm.at[pl.ds(wid * B_PER_W, B_PER_W)])


def kernel(X, y, V, w, b):
    xf = X.astype(jnp.int32).reshape(BATCH * FIELDS)
    w8 = w.reshape(w.shape[0] // WPACK, WPACK)
    b16 = jnp.broadcast_to(b.astype(jnp.float32), (GROUP,))
    mesh = plsc.VectorSubcoreMesh(core_axis_name="c", subcore_axis_name="s",
                                  num_cores=NC, num_subcores=NS)
    fm = pl.kernel(
        _fm_body,
        out_type=jax.ShapeDtypeStruct((BATCH,), jnp.float32),
        mesh=mesh,
        scratch_types=[
            pltpu.VMEM((IDX_PER_W,), jnp.int32),
            pltpu.VMEM((IDX_PER_W,), jnp.int32),
            pltpu.VMEM((ROWS_PER_G, DIM), jnp.float32),
            pltpu.VMEM((ROWS_PER_G, DIM), jnp.float32),
            pltpu.VMEM((ROWS_PER_G, DIM), jnp.float32),
            pltpu.VMEM((ROWS_PER_G, WPACK), jnp.float32),
            pltpu.VMEM((ROWS_PER_G, WPACK), jnp.float32),
            pltpu.VMEM((ROWS_PER_G, WPACK), jnp.float32),
            pltpu.VMEM((B_PER_W,), jnp.float32),
            pltpu.VMEM((GROUP,), jnp.float32),
            pltpu.SemaphoreType.DMA,
            pltpu.SemaphoreType.DMA,
            pltpu.SemaphoreType.DMA,
        ],
        compiler_params=pltpu.CompilerParams(needs_layout_passes=False,
                                             use_tc_tiling_on_sc=False),
    )
    y_pred = fm(xf, V, w8, b16).reshape(BATCH, 1)
    y_true = y.reshape(BATCH, 1)
    return (y_true, y_pred)
